# baseline (speedup 1.0000x reference)
import jax
import jax.numpy as jnp
from jax.experimental import pallas as pl
from jax.experimental.pallas import tpu as pltpu

_K = 256


def _lane_cumsum_excl(x):
    L = x.shape[1]
    lane = jax.lax.broadcasted_iota(jnp.int32, x.shape, 1)
    y = x
    sh = 1
    while sh < L:
        y = y + jnp.where(lane >= sh, pltpu.roll(y, sh, axis=1), 0)
        sh *= 2
    return y - x


def _fused_body(b_ref, a_ref, h_ref, w1_ref, wg_ref,
                hout_ref, ap_ref, idx_ref,
                ar_ref, idxv_ref, idxs_ref, sem_a, sem_i):
    N, F = h_ref.shape
    exact = jax.lax.Precision.HIGHEST

    w8 = jnp.broadcast_to(w1_ref[...], (8, F))
    z = jax.lax.dot_general(
        w8, h_ref[...],
        dimension_numbers=(((1,), (1,)), ((), ())),
        precision=exact, preferred_element_type=jnp.float32)
    s = jax.nn.sigmoid(z[0:1, :] + b_ref[0, 0])

    def bisect(_, lohi):
        lo, hi = lohi
        mid = jax.lax.shift_right_logical(lo + hi, 1)
        mid_f = pltpu.bitcast(mid, jnp.float32)
        cnt = jnp.sum((s >= mid_f).astype(jnp.int32), keepdims=True)
        ge = cnt >= _K
        return jnp.where(ge, mid, lo), jnp.where(ge, hi, mid)
    lo0 = jnp.zeros((1, 1), jnp.int32)
    hi0 = jnp.full((1, 1), 0x3F800001, jnp.int32)
    lo, _ = jax.lax.fori_loop(0, 31, bisect, (lo0, hi0))
    vk = pltpu.bitcast(lo, jnp.float32)

    gt = s > vk
    eq = s == vk
    quota = _K - jnp.sum(gt.astype(jnp.int32), keepdims=True)
    eq_pre = _lane_cumsum_excl(eq.astype(jnp.int32))
    sel = gt | (eq & (eq_pre < quota))

    seli = sel.astype(jnp.int32)
    pos = _lane_cumsum_excl(seli)
    sub_k = jax.lax.broadcasted_iota(jnp.int32, (_K, N), 0)
    ct = ((pos == sub_k) & sel).astype(jnp.float32)

    idxf8 = jax.lax.broadcasted_iota(jnp.int32, (8, N), 1).astype(jnp.float32)
    s8 = jnp.broadcast_to(s, (8, N))
    nt = (((1,), (1,)), ((), ()))
    sc8 = jax.lax.dot_general(s8, ct, nt, precision=exact,
                              preferred_element_type=jnp.float32)
    ic8 = jax.lax.dot_general(idxf8, ct, nt, precision=exact,
                              preferred_element_type=jnp.float32)
    sc_col = jax.lax.dot_general(ct, s8, nt, precision=exact,
                                 preferred_element_type=jnp.float32)
    ic_col = jax.lax.dot_general(ct, idxf8, nt, precision=exact,
                                 preferred_element_type=jnp.float32)

    g = sc8[0:1, :] > sc_col[:, 0:1]
    e = sc8[0:1, :] == sc_col[:, 0:1]
    tl = ic8[0:1, :] < ic_col[:, 0:1]
    rank = jnp.sum((g | (e & tl)).astype(jnp.int32),
                   axis=1, keepdims=True)
    lane_k = jax.lax.broadcasted_iota(jnp.int32, (_K, _K), 1)
    r_oh = (rank == lane_k).astype(jnp.float32)

    mm = (((1,), (0,)), ((), ()))
    vals8 = jax.lax.dot_general(sc8, r_oh, mm, precision=exact,
                                preferred_element_type=jnp.float32)
    idxs8 = jax.lax.dot_general(ic8, r_oh, mm, precision=exact,
                                preferred_element_type=jnp.float32)
    vals_row = vals8[0:1, :]
    idx_row = idxs8[0:1, :].astype(jnp.int32)
    idx_ref[...] = idx_row

    idxv_ref[...] = idx_row
    cp = pltpu.make_async_copy(idxv_ref, idxs_ref, sem_i)
    cp.start()
    cp.wait()

    def issue_a(k, carry):
        r = idxs_ref[0, k]
        pltpu.make_async_copy(a_ref.at[pl.ds(r, 1), :],
                              ar_ref.at[pl.ds(k, 1), :], sem_a).start()
        return carry
    jax.lax.fori_loop(0, _K, issue_a, 0, unroll=True)

    sub_n = jax.lax.broadcasted_iota(jnp.int32, (N, _K), 0)
    oh = (sub_n == idx_row).astype(jnp.float32)
    hg = jax.lax.dot_general(
        oh, h_ref[...], (((0,), (0,)), ((), ())),
        precision=exact, preferred_element_type=jnp.float32)

    pltpu.make_async_copy(a_ref.at[pl.ds(0, _K), :],
                          ar_ref.at[pl.ds(0, _K), :], sem_a).wait()

    ap = jnp.dot(ar_ref[...], oh, precision=exact,
                 preferred_element_type=jnp.float32)
    ap_ref[...] = ap

    t = jnp.dot(ap * vals_row, hg, preferred_element_type=jnp.float32)
    out = jnp.dot(t, wg_ref[...], preferred_element_type=jnp.float32)
    hout_ref[...] = jnp.maximum(out, 0.0)


def kernel(H, A, gcn_w, proj_w, proj_b):
    N, F = H.shape
    Fout = gcn_w.shape[1]
    w1 = proj_w.reshape(1, F).astype(jnp.float32)
    b11 = jnp.reshape(proj_b, (1, 1)).astype(jnp.float32)
    Hout, Ap, idx2 = pl.pallas_call(
        _fused_body,
        out_shape=(jax.ShapeDtypeStruct((_K, Fout), jnp.float32),
                   jax.ShapeDtypeStruct((_K, _K), jnp.float32),
                   jax.ShapeDtypeStruct((1, _K), jnp.int32)),
        grid=(1,),
        in_specs=[
            pl.BlockSpec((1, 1), lambda i: (0, 0),
                         memory_space=pltpu.MemorySpace.SMEM),
            pl.BlockSpec(memory_space=pltpu.MemorySpace.HBM),
            pl.BlockSpec((N, F), lambda i: (0, 0)),
            pl.BlockSpec((1, F), lambda i: (0, 0)),
            pl.BlockSpec((F, Fout), lambda i: (0, 0)),
        ],
        out_specs=(
            pl.BlockSpec((_K, Fout), lambda i: (0, 0)),
            pl.BlockSpec((_K, _K), lambda i: (0, 0)),
            pl.BlockSpec((1, _K), lambda i: (0, 0)),
        ),
        scratch_shapes=[
            pltpu.VMEM((_K, N), jnp.float32),
            pltpu.VMEM((1, _K), jnp.int32),
            pltpu.SMEM((1, _K), jnp.int32),
            pltpu.SemaphoreType.DMA,
            pltpu.SemaphoreType.DMA,
        ],
        compiler_params=pltpu.CompilerParams(
            dimension_semantics=("arbitrary",)),
    )(b11, A, H, w1, gcn_w)
    return Hout, Ap, idx2.reshape(_K)


# speedup vs baseline: 1.2882x; 1.2882x over previous
import jax
import jax.numpy as jnp
from jax.experimental import pallas as pl
from jax.experimental.pallas import tpu as pltpu

_K = 256
_KH = _K // 2


def _scores_body(h_ref, w_ref, b_ref, o_ref):
    w8 = jnp.broadcast_to(w_ref[...], (8, w_ref.shape[1]))
    z = jax.lax.dot_general(
        w8, h_ref[...],
        dimension_numbers=(((1,), (1,)), ((), ())),
        precision=jax.lax.Precision.HIGHEST,
        preferred_element_type=jnp.float32)
    o_ref[...] = jax.nn.sigmoid(z[0:1, :] + b_ref[0, 0])


def _scores(H, proj_w, proj_b):
    N, F = H.shape
    w1 = proj_w.reshape(1, F).astype(jnp.float32)
    b11 = jnp.reshape(proj_b, (1, 1)).astype(jnp.float32)
    tm = 512 if N % 512 == 0 else N
    out = pl.pallas_call(
        _scores_body,
        out_shape=jax.ShapeDtypeStruct((1, N), jnp.float32),
        grid=(N // tm,),
        in_specs=[
            pl.BlockSpec((tm, F), lambda i: (i, 0)),
            pl.BlockSpec((1, F), lambda i: (0, 0)),
            pl.BlockSpec((1, 1), lambda i: (0, 0),
                         memory_space=pltpu.MemorySpace.SMEM),
        ],
        out_specs=pl.BlockSpec((1, tm), lambda i: (0, i)),
        compiler_params=pltpu.CompilerParams(
            dimension_semantics=("parallel",)),
    )(H, w1, b11)
    return out[0]


def _pooled_body(idx_ref, a_ref, h_ref, idxr_ref, vals_ref, w_ref,
                 hout_ref, ap_ref, ar_ref, hg_ref, oh_ref, sem_a, sem_h):
    base = pl.program_id(0) * _KH

    def issue_a(k, carry):
        r = idx_ref[base + k]
        pltpu.make_async_copy(a_ref.at[pl.ds(r, 1), :],
                              ar_ref.at[pl.ds(k, 1), :], sem_a).start()
        return carry
    jax.lax.fori_loop(0, _KH, issue_a, 0, unroll=True)

    def issue_h(k, carry):
        r = idx_ref[k]
        pltpu.make_async_copy(h_ref.at[pl.ds(r, 1), :],
                              hg_ref.at[pl.ds(k, 1), :], sem_h).start()
        return carry
    jax.lax.fori_loop(0, _K, issue_h, 0, unroll=True)

    n, k = oh_ref.shape
    sub_ids = jax.lax.broadcasted_iota(jnp.int32, (n, k), 0)
    oh_ref[...] = (sub_ids == idxr_ref[...]).astype(jnp.float32)

    pltpu.make_async_copy(a_ref.at[pl.ds(0, _KH), :],
                          ar_ref.at[pl.ds(0, _KH), :], sem_a).wait()
    pltpu.make_async_copy(h_ref.at[pl.ds(0, _K), :],
                          hg_ref.at[pl.ds(0, _K), :], sem_h).wait()

    ap = jnp.dot(ar_ref[...], oh_ref[...],
                 precision=jax.lax.Precision.HIGHEST,
                 preferred_element_type=jnp.float32)
    ap_ref[...] = ap

    t = jnp.dot(ap * vals_ref[...], hg_ref[...],
                preferred_element_type=jnp.float32)
    out = jnp.dot(t, w_ref[...], preferred_element_type=jnp.float32)
    hout_ref[...] = jnp.maximum(out, 0.0)


def _pooled_gcn(idx, vals, A, H, Wg):
    N, F = H.shape
    Fout = Wg.shape[1]
    idx_row = idx.reshape(1, _K).astype(jnp.int32)
    vals_row = vals.reshape(1, _K).astype(jnp.float32)
    grid_spec = pltpu.PrefetchScalarGridSpec(
        num_scalar_prefetch=1,
        grid=(2,),
        in_specs=[
            pl.BlockSpec(memory_space=pltpu.MemorySpace.HBM),
            pl.BlockSpec(memory_space=pltpu.MemorySpace.HBM),
            pl.BlockSpec((1, _K), lambda i, idx_ref: (0, 0)),
            pl.BlockSpec((1, _K), lambda i, idx_ref: (0, 0)),
            pl.BlockSpec((F, Fout), lambda i, idx_ref: (0, 0)),
        ],
        out_specs=[
            pl.BlockSpec((_KH, Fout), lambda i, idx_ref: (i, 0)),
            pl.BlockSpec((_KH, _K), lambda i, idx_ref: (i, 0)),
        ],
        scratch_shapes=[
            pltpu.VMEM((_KH, N), jnp.float32),
            pltpu.VMEM((_K, F), jnp.float32),
            pltpu.VMEM((N, _K), jnp.float32),
            pltpu.SemaphoreType.DMA,
            pltpu.SemaphoreType.DMA,
        ],
    )
    return pl.pallas_call(
        _pooled_body,
        grid_spec=grid_spec,
        out_shape=(jax.ShapeDtypeStruct((_K, Fout), jnp.float32),
                   jax.ShapeDtypeStruct((_K, _K), jnp.float32)),
        compiler_params=pltpu.CompilerParams(
            dimension_semantics=("parallel",)),
    )(idx, A, H, idx_row, vals_row, Wg)


def kernel(H, A, gcn_w, proj_w, proj_b):
    N, F = H.shape
    scores = _scores(H, proj_w, proj_b)
    vals, idx = jax.lax.top_k(scores, _K)
    Hout, Ap = _pooled_gcn(idx, vals, A, H, gcn_w)
    return Hout, Ap, idx
